# Initial kernel scaffold; baseline (speedup 1.0000x reference)
import jax
import jax.numpy as jnp
from jax.experimental import pallas as pl


def kernel(embeddings, w, b):
    raise NotImplementedError("write your pallas kernel here")


# speedup vs baseline: 2.6341x; 2.6341x over previous
import functools

import jax
import jax.numpy as jnp
from jax.experimental import pallas as pl
from jax.experimental.pallas import tpu as pltpu

_B, _N, _M, _D = 8, 256, 16, 256


def _loss_kernel(w_ref, b_ref, e_ref, o_ref):
    n, m, d = _N, _M, _D
    nm = n * m
    w = w_ref[0]
    b = b_ref[0]

    e4 = e_ref[0]
    e = e4.reshape(nm, d)

    c = jnp.sum(e4, axis=1) * (1.0 / m)

    dots = jax.lax.dot_general(
        e, c, (((1,), (1,)), ((), ())),
        preferred_element_type=jnp.float32)
    s = w * dots + b

    sq = jnp.sum(e * e, axis=1, keepdims=True)

    rows = jax.lax.broadcasted_iota(jnp.int32, (nm, n), 0)
    cols = jax.lax.broadcasted_iota(jnp.int32, (nm, n), 1)
    mask = cols == (rows // m)
    s_mod = jnp.where(mask, s + (s - w * sq - b) * (1.0 / (m - 1)), s)

    mx = jnp.max(s_mod, axis=1, keepdims=True)
    ssum = jnp.sum(jnp.exp(s_mod - mx), axis=1, keepdims=True)
    lse = mx + jnp.log(ssum)
    self_vals = jnp.where(mask, s_mod, 0.0)
    partial = jnp.sum(lse) - jnp.sum(self_vals)
    o_ref[...] = jnp.full((1, 8, 128), partial, jnp.float32)


@functools.partial(jax.jit, static_argnames=())
def kernel(embeddings, w, b):
    bsz, n, m, d = embeddings.shape
    w1 = jnp.reshape(w.astype(jnp.float32), (1,))
    b1 = jnp.reshape(b.astype(jnp.float32), (1,))
    partials = pl.pallas_call(
        _loss_kernel,
        grid=(bsz,),
        in_specs=[
            pl.BlockSpec(memory_space=pltpu.SMEM),
            pl.BlockSpec(memory_space=pltpu.SMEM),
            pl.BlockSpec((1, n, m, d), lambda i: (i, 0, 0, 0)),
        ],
        out_specs=pl.BlockSpec((1, 8, 128), lambda i: (i, 0, 0)),
        out_shape=jax.ShapeDtypeStruct((bsz, 8, 128), jnp.float32),
        compiler_params=pltpu.CompilerParams(
            dimension_semantics=("parallel",),
            vmem_limit_bytes=100 * 1024 * 1024,
        ),
    )(w1, b1, embeddings)
    return jnp.sum(partials[:, 0, 0])
